# baseline (speedup 1.0000x reference)
import functools

import jax
import jax.numpy as jnp
from jax import lax
from jax.experimental import pallas as pl
from jax.experimental.pallas import tpu as pltpu

N = 100000
D = 128
G = 1024
B = 2000
W = 32
NBLK = N // B


def _fused_kernel(first_ref, nwin_ref, h_ref, ids_ref,
                  w1a_ref, b1a_ref, w1b_ref, b1b_ref,
                  w2a_ref, b2a_ref, w2b_ref, b2b_ref,
                  out_ref, acc_ref):
    i = pl.program_id(0)

    @pl.when(i == 0)
    def _init():
        acc_ref[...] = jnp.zeros_like(acc_ref)

    hb = h_ref[...].astype(jnp.bfloat16)
    x = jnp.dot(hb, w1a_ref[...].astype(jnp.bfloat16),
                preferred_element_type=jnp.float32)
    x = jnp.maximum(x + b1a_ref[...], 0.0).astype(jnp.bfloat16)
    act = jnp.dot(x, w1b_ref[...].astype(jnp.bfloat16),
                  preferred_element_type=jnp.float32)
    act = jnp.maximum(act + b1b_ref[...], 0.0)

    ids = ids_ref[0, 0, :]
    first = first_ref[i]
    nwin = nwin_ref[i]

    col_iota = lax.broadcasted_iota(jnp.int32, (B, W), 1)

    def body(k, carry):
        base = first + k * W
        rel = ids - base
        oh = (rel[:, None] == col_iota).astype(jnp.float32)
        partial = lax.dot_general(
            oh, act, (((0,), (0,)), ((), ())),
            preferred_element_type=jnp.float32)
        acc_ref[pl.ds(base, W), :] += partial
        return carry

    lax.fori_loop(0, nwin, body, 0)

    @pl.when(i == NBLK - 1)
    def _finish():
        pooled = acc_ref[0:G, :]
        y = jnp.dot(pooled, w2a_ref[...], preferred_element_type=jnp.float32)
        y = jnp.maximum(y + b2a_ref[...], 0.0)
        z = jnp.dot(y, w2b_ref[...], preferred_element_type=jnp.float32)
        out_ref[...] = jnp.maximum(z + b2b_ref[...], 0.0)


@jax.jit
def kernel(h, graph_ids, W1a, b1a, W1b, b1b, W2a, b2a, W2b, b2b):
    ids32 = graph_ids.astype(jnp.int32)
    ids3 = ids32.reshape(NBLK, 1, B)
    firsts = ids32[::B]
    lasts = ids32[B - 1::B]
    nwin = (lasts - firsts) // W + 1

    full = lambda shape: pl.BlockSpec(shape, lambda i, *_: (0,) * len(shape))
    row = lambda: pl.BlockSpec((1, D), lambda i, *_: (0, 0))

    grid_spec = pltpu.PrefetchScalarGridSpec(
        num_scalar_prefetch=2,
        grid=(NBLK,),
        in_specs=[
            pl.BlockSpec((B, D), lambda i, *_: (i, 0)),
            pl.BlockSpec((1, 1, B), lambda i, *_: (i, 0, 0)),
            full((D, D)), row(), full((D, D)), row(),
            full((D, D)), row(), full((D, D)), row(),
        ],
        out_specs=pl.BlockSpec((G, D), lambda i, *_: (0, 0)),
        scratch_shapes=[pltpu.VMEM((G + W, D), jnp.float32)],
    )

    return pl.pallas_call(
        _fused_kernel,
        grid_spec=grid_spec,
        out_shape=jax.ShapeDtypeStruct((G, D), jnp.float32),
    )(firsts, nwin, h, ids3,
      W1a, b1a.reshape(1, D), W1b, b1b.reshape(1, D),
      W2a, b2a.reshape(1, D), W2b, b2b.reshape(1, D))


# speedup vs baseline: 1.1161x; 1.1161x over previous
import jax
import jax.numpy as jnp
from jax import lax
from jax.experimental import pallas as pl
from jax.experimental.pallas import tpu as pltpu

N = 100000
D = 128
G = 1024
B = 2000
W = 32
NBLK = N // B
NCORE = 2
NB = NBLK // NCORE


def _mlp_pool_kernel(first_ref, nwin_ref, h_ref, ids_ref,
                     w1a_ref, b1a_ref, w1b_ref, b1b_ref,
                     out_ref, acc_ref):
    c = pl.program_id(0)
    j = pl.program_id(1)

    @pl.when(j == 0)
    def _init():
        acc_ref[...] = jnp.zeros_like(acc_ref)

    x = jnp.dot(h_ref[...], w1a_ref[...], preferred_element_type=jnp.float32)
    x = jnp.maximum(x + b1a_ref[...], 0.0)
    act = jnp.dot(x, w1b_ref[...], preferred_element_type=jnp.float32)
    act = jnp.maximum(act + b1b_ref[...], 0.0)

    ids = ids_ref[0, 0, :]
    i = c * NB + j
    first = first_ref[i]
    nwin = nwin_ref[i]

    col_iota = lax.broadcasted_iota(jnp.int32, (B, W), 1)

    def body(k, carry):
        base = first + k * W
        rel = ids - base
        oh = (rel[:, None] == col_iota).astype(jnp.float32)
        partial = lax.dot_general(
            oh, act, (((0,), (0,)), ((), ())),
            preferred_element_type=jnp.float32)
        acc_ref[pl.ds(base, W), :] += partial
        return carry

    lax.fori_loop(0, nwin, body, 0)

    @pl.when(j == NB - 1)
    def _finish():
        out_ref[0] = acc_ref[0:G, :]


def _combine_kernel(p_ref, w2a_ref, b2a_ref, w2b_ref, b2b_ref, out_ref):
    pooled = p_ref[0] + p_ref[1]
    y = jnp.dot(pooled, w2a_ref[...], preferred_element_type=jnp.float32)
    y = jnp.maximum(y + b2a_ref[...], 0.0)
    z = jnp.dot(y, w2b_ref[...], preferred_element_type=jnp.float32)
    out_ref[...] = jnp.maximum(z + b2b_ref[...], 0.0)


@jax.jit
def kernel(h, graph_ids, W1a, b1a, W1b, b1b, W2a, b2a, W2b, b2b):
    ids32 = graph_ids.astype(jnp.int32)
    ids3 = ids32.reshape(NBLK, 1, B)
    firsts = ids32[::B]
    lasts = ids32[B - 1::B]
    nwin = (lasts - firsts) // W + 1

    full = lambda shape: pl.BlockSpec(shape, lambda c, j, *_: (0,) * len(shape))
    row = lambda: pl.BlockSpec((1, D), lambda c, j, *_: (0, 0))

    grid_spec = pltpu.PrefetchScalarGridSpec(
        num_scalar_prefetch=2,
        grid=(NCORE, NB),
        in_specs=[
            pl.BlockSpec((B, D), lambda c, j, *_: (c * NB + j, 0)),
            pl.BlockSpec((1, 1, B), lambda c, j, *_: (c * NB + j, 0, 0)),
            full((D, D)), row(), full((D, D)), row(),
        ],
        out_specs=pl.BlockSpec((1, G, D), lambda c, j, *_: (c, 0, 0)),
        scratch_shapes=[pltpu.VMEM((G + W, D), jnp.float32)],
    )

    partials = pl.pallas_call(
        _mlp_pool_kernel,
        grid_spec=grid_spec,
        out_shape=jax.ShapeDtypeStruct((NCORE, G, D), jnp.float32),
        compiler_params=pltpu.CompilerParams(
            dimension_semantics=("parallel", "arbitrary")),
    )(firsts, nwin, h, ids3,
      W1a, b1a.reshape(1, D), W1b, b1b.reshape(1, D))

    return pl.pallas_call(
        _combine_kernel,
        in_specs=[
            pl.BlockSpec((NCORE, G, D), lambda: (0, 0, 0)),
            pl.BlockSpec((D, D), lambda: (0, 0)),
            pl.BlockSpec((1, D), lambda: (0, 0)),
            pl.BlockSpec((D, D), lambda: (0, 0)),
            pl.BlockSpec((1, D), lambda: (0, 0)),
        ],
        out_specs=pl.BlockSpec((G, D), lambda: (0, 0)),
        out_shape=jax.ShapeDtypeStruct((G, D), jnp.float32),
    )(partials, W2a, b2a.reshape(1, D), W2b, b2b.reshape(1, D))


# speedup vs baseline: 1.4069x; 1.2606x over previous
import jax
import jax.numpy as jnp
from jax import lax
from jax.experimental import pallas as pl
from jax.experimental.pallas import tpu as pltpu

N = 100000
D = 128
G = 1024
B = 2000
W = 64
NBLK = N // B


def _step(i, first_ref, nwin_ref, h_ref, ids_ref,
          w1a_ref, b1a_ref, w1b_ref, b1b_ref,
          acc_ref, write_ref, read_ref):
    x = jnp.dot(h_ref[...], w1a_ref[...], preferred_element_type=jnp.float32)
    x = jnp.maximum(x + b1a_ref[...], 0.0)
    act = jnp.dot(x, w1b_ref[...], preferred_element_type=jnp.float32)
    write_ref[...] = jnp.maximum(act + b1b_ref[...], 0.0)

    ids = ids_ref[0, 0, :]
    first = first_ref[i]
    nwin = nwin_ref[i]

    col_iota = lax.broadcasted_iota(jnp.int32, (B, W), 1)
    act_prev = read_ref[...]

    def window(k):
        base = first + k * W
        rel = ids - base
        oh = (rel[:, None] == col_iota).astype(jnp.float32)
        partial = lax.dot_general(
            oh, act_prev, (((0,), (0,)), ((), ())),
            preferred_element_type=jnp.float32)
        acc_ref[pl.ds(base, W), :] += partial

    window(0)

    def body(k, carry):
        window(k)
        return carry

    lax.fori_loop(1, nwin, body, 0)


def _fused_kernel(first_ref, nwin_ref, h_ref, ids_ref,
                  w1a_ref, b1a_ref, w1b_ref, b1b_ref,
                  w2a_ref, b2a_ref, w2b_ref, b2b_ref,
                  out_ref, acc_ref, act0_ref, act1_ref):
    i = pl.program_id(0)

    @pl.when(i == 0)
    def _init():
        acc_ref[...] = jnp.zeros_like(acc_ref)

    args = (i, first_ref, nwin_ref, h_ref, ids_ref,
            w1a_ref, b1a_ref, w1b_ref, b1b_ref, acc_ref)

    @pl.when(i % 2 == 0)
    def _even():
        _step(*args, act0_ref, act1_ref)

    @pl.when(i % 2 == 1)
    def _odd():
        _step(*args, act1_ref, act0_ref)

    @pl.when(i == NBLK)
    def _finish():
        pooled = acc_ref[0:G, :]
        y = jnp.dot(pooled, w2a_ref[...], preferred_element_type=jnp.float32)
        y = jnp.maximum(y + b2a_ref[...], 0.0)
        z = jnp.dot(y, w2b_ref[...], preferred_element_type=jnp.float32)
        out_ref[...] = jnp.maximum(z + b2b_ref[...], 0.0)


@jax.jit
def kernel(h, graph_ids, W1a, b1a, W1b, b1b, W2a, b2a, W2b, b2b):
    ids32 = graph_ids.astype(jnp.int32)
    ids3 = ids32.reshape(NBLK, 1, B)
    firsts = ids32[::B]
    lasts = ids32[B - 1::B]
    nwin = (lasts - firsts) // W + 1
    sfirst = jnp.concatenate([jnp.full((1,), G, jnp.int32), firsts])
    snwin = jnp.concatenate([jnp.ones((1,), jnp.int32), nwin])

    full = lambda shape: pl.BlockSpec(shape, lambda i, *_: (0,) * len(shape))
    row = lambda: pl.BlockSpec((1, D), lambda i, *_: (0, 0))

    last_blk = NBLK - 1
    grid_spec = pltpu.PrefetchScalarGridSpec(
        num_scalar_prefetch=2,
        grid=(NBLK + 1,),
        in_specs=[
            pl.BlockSpec((B, D),
                         lambda i, *_: (jnp.minimum(i, last_blk), 0)),
            pl.BlockSpec((1, 1, B),
                         lambda i, *_: (jnp.maximum(i - 1, 0), 0, 0)),
            full((D, D)), row(), full((D, D)), row(),
            full((D, D)), row(), full((D, D)), row(),
        ],
        out_specs=pl.BlockSpec((G, D), lambda i, *_: (0, 0)),
        scratch_shapes=[
            pltpu.VMEM((G + W, D), jnp.float32),
            pltpu.VMEM((B, D), jnp.float32),
            pltpu.VMEM((B, D), jnp.float32),
        ],
    )

    return pl.pallas_call(
        _fused_kernel,
        grid_spec=grid_spec,
        out_shape=jax.ShapeDtypeStruct((G, D), jnp.float32),
    )(sfirst, snwin, h, ids3,
      W1a, b1a.reshape(1, D), W1b, b1b.reshape(1, D),
      W2a, b2a.reshape(1, D), W2b, b2b.reshape(1, D))


# speedup vs baseline: 1.4238x; 1.0120x over previous
import jax
import jax.numpy as jnp
from jax import lax
from jax.experimental import pallas as pl
from jax.experimental.pallas import tpu as pltpu

N = 100000
D = 128
G = 1024
B = 2000
W = 64
NBLK = N // B


def _step(i, first_ref, nwin_ref, h_ref, ids_ref,
          w1a_ref, b1a_ref, w1b_ref, b1b_ref,
          acc_ref, write_ref, read_ref):
    x = jnp.dot(h_ref[...], w1a_ref[...], preferred_element_type=jnp.float32)
    x = jnp.maximum(x + b1a_ref[...], 0.0)
    act = jnp.dot(x, w1b_ref[...], preferred_element_type=jnp.float32)
    write_ref[...] = jnp.maximum(act + b1b_ref[...], 0.0)

    ids_row = ids_ref[0]
    first = first_ref[i]
    nwin = nwin_ref[i]

    row_iota = lax.broadcasted_iota(jnp.int32, (W, B), 0)
    act_prev = read_ref[...]

    def window(k):
        base = first + k * W
        rel = ids_row - base
        oh_t = (rel == row_iota).astype(jnp.float32)
        partial = jnp.dot(oh_t, act_prev,
                          preferred_element_type=jnp.float32)
        acc_ref[pl.ds(base, W), :] += partial

    window(0)

    def body(k, carry):
        window(k)
        return carry

    lax.fori_loop(1, nwin, body, 0)


def _fused_kernel(first_ref, nwin_ref, h_ref, ids_ref,
                  w1a_ref, b1a_ref, w1b_ref, b1b_ref,
                  w2a_ref, b2a_ref, w2b_ref, b2b_ref,
                  out_ref, acc_ref, act0_ref, act1_ref):
    i = pl.program_id(0)

    @pl.when(i == 0)
    def _init():
        acc_ref[...] = jnp.zeros_like(acc_ref)

    args = (i, first_ref, nwin_ref, h_ref, ids_ref,
            w1a_ref, b1a_ref, w1b_ref, b1b_ref, acc_ref)

    @pl.when(i % 2 == 0)
    def _even():
        _step(*args, act0_ref, act1_ref)

    @pl.when(i % 2 == 1)
    def _odd():
        _step(*args, act1_ref, act0_ref)

    @pl.when(i == NBLK)
    def _finish():
        pooled = acc_ref[0:G, :]
        y = jnp.dot(pooled, w2a_ref[...], preferred_element_type=jnp.float32)
        y = jnp.maximum(y + b2a_ref[...], 0.0)
        z = jnp.dot(y, w2b_ref[...], preferred_element_type=jnp.float32)
        out_ref[...] = jnp.maximum(z + b2b_ref[...], 0.0)


@jax.jit
def kernel(h, graph_ids, W1a, b1a, W1b, b1b, W2a, b2a, W2b, b2b):
    ids32 = graph_ids.astype(jnp.int32)
    ids3 = ids32.reshape(NBLK, 1, B)
    firsts = ids32[::B]
    lasts = ids32[B - 1::B]
    nwin = (lasts - firsts) // W + 1
    sfirst = jnp.concatenate([jnp.full((1,), G, jnp.int32), firsts])
    snwin = jnp.concatenate([jnp.ones((1,), jnp.int32), nwin])

    full = lambda shape: pl.BlockSpec(shape, lambda i, *_: (0,) * len(shape))
    row = lambda: pl.BlockSpec((1, D), lambda i, *_: (0, 0))

    last_blk = NBLK - 1
    grid_spec = pltpu.PrefetchScalarGridSpec(
        num_scalar_prefetch=2,
        grid=(NBLK + 1,),
        in_specs=[
            pl.BlockSpec((B, D),
                         lambda i, *_: (jnp.minimum(i, last_blk), 0)),
            pl.BlockSpec((1, 1, B),
                         lambda i, *_: (jnp.maximum(i - 1, 0), 0, 0)),
            full((D, D)), row(), full((D, D)), row(),
            full((D, D)), row(), full((D, D)), row(),
        ],
        out_specs=pl.BlockSpec((G, D), lambda i, *_: (0, 0)),
        scratch_shapes=[
            pltpu.VMEM((G + W, D), jnp.float32),
            pltpu.VMEM((B, D), jnp.float32),
            pltpu.VMEM((B, D), jnp.float32),
        ],
    )

    return pl.pallas_call(
        _fused_kernel,
        grid_spec=grid_spec,
        out_shape=jax.ShapeDtypeStruct((G, D), jnp.float32),
    )(sfirst, snwin, h, ids3,
      W1a, b1a.reshape(1, D), W1b, b1b.reshape(1, D),
      W2a, b2a.reshape(1, D), W2b, b2b.reshape(1, D))


# speedup vs baseline: 1.9365x; 1.3601x over previous
import jax
import jax.numpy as jnp
from jax import lax
from jax.experimental import pallas as pl
from jax.experimental.pallas import tpu as pltpu

N = 100000
D = 128
G = 1024
B = 4000
W = 64
NBLK = N // B


def _step(i, first_ref, nwin_ref, h_ref, ids_ref,
          w1a_ref, b1a_ref, w1b_ref, b1b_ref,
          acc_ref, write_ref, read_ref):
    x = jnp.dot(h_ref[...], w1a_ref[...], preferred_element_type=jnp.float32)
    x = jnp.maximum(x + b1a_ref[...], 0.0)
    act = jnp.dot(x, w1b_ref[...], preferred_element_type=jnp.float32)
    write_ref[...] = jnp.maximum(act + b1b_ref[...], 0.0)

    ids_row = ids_ref[0]
    first = first_ref[i]
    nwin = nwin_ref[i]

    row_iota = lax.broadcasted_iota(jnp.int32, (W, B), 0)
    act_prev = read_ref[...]

    def window(k):
        base = first + k * W
        rel = ids_row - base
        oh_t = (rel == row_iota).astype(jnp.float32)
        partial = jnp.dot(oh_t, act_prev,
                          preferred_element_type=jnp.float32)
        acc_ref[pl.ds(base, W), :] += partial

    window(0)

    def body(k, carry):
        window(k)
        return carry

    lax.fori_loop(1, nwin, body, 0)


def _fused_kernel(first_ref, nwin_ref, h_ref, ids_ref,
                  w1a_ref, b1a_ref, w1b_ref, b1b_ref,
                  w2a_ref, b2a_ref, w2b_ref, b2b_ref,
                  out_ref, acc_ref, act0_ref, act1_ref):
    i = pl.program_id(0)

    @pl.when(i == 0)
    def _init():
        acc_ref[...] = jnp.zeros_like(acc_ref)

    args = (i, first_ref, nwin_ref, h_ref, ids_ref,
            w1a_ref, b1a_ref, w1b_ref, b1b_ref, acc_ref)

    @pl.when(i % 2 == 0)
    def _even():
        _step(*args, act0_ref, act1_ref)

    @pl.when(i % 2 == 1)
    def _odd():
        _step(*args, act1_ref, act0_ref)

    @pl.when(i == NBLK)
    def _finish():
        pooled = acc_ref[0:G, :]
        y = jnp.dot(pooled, w2a_ref[...], preferred_element_type=jnp.float32)
        y = jnp.maximum(y + b2a_ref[...], 0.0)
        z = jnp.dot(y, w2b_ref[...], preferred_element_type=jnp.float32)
        out_ref[...] = jnp.maximum(z + b2b_ref[...], 0.0)


@jax.jit
def kernel(h, graph_ids, W1a, b1a, W1b, b1b, W2a, b2a, W2b, b2b):
    ids32 = graph_ids.astype(jnp.int32)
    ids3 = ids32.reshape(NBLK, 1, B)
    firsts = ids32[::B]
    lasts = ids32[B - 1::B]
    nwin = (lasts - firsts) // W + 1
    sfirst = jnp.concatenate([jnp.full((1,), G, jnp.int32), firsts])
    snwin = jnp.concatenate([jnp.ones((1,), jnp.int32), nwin])

    full = lambda shape: pl.BlockSpec(shape, lambda i, *_: (0,) * len(shape))
    row = lambda: pl.BlockSpec((1, D), lambda i, *_: (0, 0))

    last_blk = NBLK - 1
    grid_spec = pltpu.PrefetchScalarGridSpec(
        num_scalar_prefetch=2,
        grid=(NBLK + 1,),
        in_specs=[
            pl.BlockSpec((B, D),
                         lambda i, *_: (jnp.minimum(i, last_blk), 0)),
            pl.BlockSpec((1, 1, B),
                         lambda i, *_: (jnp.maximum(i - 1, 0), 0, 0)),
            full((D, D)), row(), full((D, D)), row(),
            full((D, D)), row(), full((D, D)), row(),
        ],
        out_specs=pl.BlockSpec((G, D), lambda i, *_: (0, 0)),
        scratch_shapes=[
            pltpu.VMEM((G + W, D), jnp.float32),
            pltpu.VMEM((B, D), jnp.float32),
            pltpu.VMEM((B, D), jnp.float32),
        ],
    )

    return pl.pallas_call(
        _fused_kernel,
        grid_spec=grid_spec,
        out_shape=jax.ShapeDtypeStruct((G, D), jnp.float32),
    )(sfirst, snwin, h, ids3,
      W1a, b1a.reshape(1, D), W1b, b1b.reshape(1, D),
      W2a, b2a.reshape(1, D), W2b, b2b.reshape(1, D))
